# Initial kernel scaffold; baseline (speedup 1.0000x reference)
import jax
import jax.numpy as jnp
from jax.experimental import pallas as pl


def kernel(cat_encoded_wg, shape_input_features_in, label_mask, device):
    raise NotImplementedError("write your pallas kernel here")


# speedup vs baseline: 845.7739x; 845.7739x over previous
import functools

import jax
import jax.numpy as jnp
from jax import lax
from jax.experimental import pallas as pl
from jax.experimental.pallas import tpu as pltpu
from jax.experimental.pallas import tpu_sc as plsc


def _unpool(table_flat, mask_flat, B, C, N, HW):
  info = plsc.get_sparse_core_info()
  NC, NS, L = info.num_cores, info.num_subcores, info.num_lanes
  NW = NC * NS
  CPW = (B * C) // NW
  TPX = 2048
  NT = HW // TPX
  GP = TPX // L

  mesh = plsc.VectorSubcoreMesh(core_axis_name="c", subcore_axis_name="s")

  @functools.partial(
      pl.kernel,
      mesh=mesh,
      compiler_params=pltpu.CompilerParams(needs_layout_passes=False),
      out_type=jax.ShapeDtypeStruct((B * C * HW,), jnp.float32),
      scratch_types=[
          pltpu.VMEM((CPW * N,), jnp.float32),
          pltpu.VMEM((TPX,), jnp.int32),
          pltpu.VMEM((CPW * TPX,), jnp.float32),
      ],
  )
  def k(table_hbm, mask_hbm, out_hbm, tab_v, idx_v, out_v):
    wid = lax.axis_index("s") * NC + lax.axis_index("c")
    row0 = wid * CPW
    b = row0 // C
    pltpu.sync_copy(table_hbm.at[pl.ds(row0 * N, CPW * N)], tab_v)

    def tile_body(t, carry):
      base = t * TPX
      pltpu.sync_copy(mask_hbm.at[pl.ds(b * HW + base, TPX)], idx_v)

      def grp(j, carry2):
        idx = idx_v[pl.ds(j * L, L)]
        for cc in range(CPW):
          idxc = idx + jnp.full((L,), cc * N, jnp.int32)
          out_v[pl.ds(cc * TPX + j * L, L)] = plsc.load_gather(tab_v, [idxc])
        return carry2

      lax.fori_loop(0, GP, grp, 0, unroll=2)
      for cc in range(CPW):
        pltpu.sync_copy(
            out_v.at[pl.ds(cc * TPX, TPX)],
            out_hbm.at[pl.ds((row0 + cc) * HW + base, TPX)],
        )
      return carry

    lax.fori_loop(0, NT, tile_body, 0)

  return k(table_flat, mask_flat)


def kernel(cat_encoded_wg, shape_input_features_in, label_mask, device):
  B, C, N = cat_encoded_wg.shape
  _, _, H, W = label_mask.shape
  HW = H * W
  out = _unpool(cat_encoded_wg.reshape(-1), label_mask.reshape(-1), B, C, N, HW)
  return out.reshape(B, C, H, W)


# speedup vs baseline: 1081.1133x; 1.2783x over previous
import functools

import jax
import jax.numpy as jnp
from jax import lax
from jax.experimental import pallas as pl
from jax.experimental.pallas import tpu as pltpu
from jax.experimental.pallas import tpu_sc as plsc


def _unpool(table_flat, mask_flat, B, C, N, HW):
  info = plsc.get_sparse_core_info()
  NC, NS, L = info.num_cores, info.num_subcores, info.num_lanes
  NW = NC * NS
  CPW = (B * C) // NW
  TPX = 4096
  NT = HW // TPX
  GP = TPX // L

  mesh = plsc.VectorSubcoreMesh(core_axis_name="c", subcore_axis_name="s")

  @functools.partial(
      pl.kernel,
      mesh=mesh,
      compiler_params=pltpu.CompilerParams(needs_layout_passes=False),
      out_type=jax.ShapeDtypeStruct((B * C * HW,), jnp.float32),
      scratch_types=[
          pltpu.VMEM((CPW * N,), jnp.float32),
          pltpu.VMEM((TPX,), jnp.int32),
          pltpu.VMEM((TPX,), jnp.int32),
          pltpu.VMEM((CPW * TPX,), jnp.float32),
          pltpu.VMEM((CPW * TPX,), jnp.float32),
          pltpu.SemaphoreType.DMA,
          pltpu.SemaphoreType.DMA,
          pltpu.SemaphoreType.DMA,
          pltpu.SemaphoreType.DMA,
      ],
  )
  def k(table_hbm, mask_hbm, out_hbm, tab_v, idx0, idx1, out0, out1,
        si0, si1, so0, so1):
    wid = lax.axis_index("s") * NC + lax.axis_index("c")
    row0 = wid * CPW
    b = row0 // C
    mbase = b * HW
    pltpu.sync_copy(table_hbm.at[pl.ds(row0 * N, CPW * N)], tab_v)

    idx_bufs = (idx0, idx1)
    out_bufs = (out0, out1)
    isems = (si0, si1)
    osems = (so0, so1)

    pltpu.async_copy(mask_hbm.at[pl.ds(mbase, TPX)], idx0, si0)
    pltpu.async_copy(mask_hbm.at[pl.ds(mbase + TPX, TPX)], idx1, si1)

    def body(tt, carry):
      for par in range(2):
        t = 2 * tt + par
        base = t * TPX
        idx_v = idx_bufs[par]
        out_v = out_bufs[par]

        pltpu.make_async_copy(mask_hbm.at[pl.ds(0, TPX)], idx_v,
                              isems[par]).wait()

        @pl.when(tt >= 1)
        def _():
          pltpu.make_async_copy(out_v, out_hbm.at[pl.ds(0, CPW * TPX)],
                                osems[par]).wait()

        def grp(j, carry2):
          idx = idx_v[pl.ds(j * L, L)]
          for cc in range(CPW):
            out_v[pl.ds(cc * TPX + j * L, L)] = plsc.load_gather(
                tab_v.at[pl.ds(cc * N, N)], [idx])
          return carry2

        lax.fori_loop(0, GP, grp, 0, unroll=4)

        @pl.when(t + 2 < NT)
        def _():
          pltpu.async_copy(mask_hbm.at[pl.ds(mbase + (t + 2) * TPX, TPX)],
                           idx_v, isems[par])

        for cc in range(CPW):
          pltpu.async_copy(
              out_v.at[pl.ds(cc * TPX, TPX)],
              out_hbm.at[pl.ds((row0 + cc) * HW + base, TPX)],
              osems[par])
      return carry

    lax.fori_loop(0, NT // 2, body, 0)

    for par in range(2):
      pltpu.make_async_copy(out_bufs[par], out_hbm.at[pl.ds(0, CPW * TPX)],
                            osems[par]).wait()

  return k(table_flat, mask_flat)


def kernel(cat_encoded_wg, shape_input_features_in, label_mask, device):
  B, C, N = cat_encoded_wg.shape
  _, _, H, W = label_mask.shape
  HW = H * W
  out = _unpool(cat_encoded_wg.reshape(-1), label_mask.reshape(-1), B, C, N, HW)
  return out.reshape(B, C, H, W)


# speedup vs baseline: 2952.5619x; 2.7310x over previous
import functools

import jax
import jax.numpy as jnp
from jax import lax
from jax.experimental import pallas as pl
from jax.experimental.pallas import tpu as pltpu
from jax.experimental.pallas import tpu_sc as plsc


def _unpool(table_flat, mask3, B, C, N, H, W):
  info = plsc.get_sparse_core_info()
  NC, NS, L = info.num_cores, info.num_subcores, info.num_lanes
  NW = NC * NS
  CPW = (B * C) // NW
  HB = 8
  TPX = HB * W
  NT = H // HB
  GPR = W // L

  mesh = plsc.VectorSubcoreMesh(core_axis_name="c", subcore_axis_name="s")

  @functools.partial(
      pl.kernel,
      mesh=mesh,
      compiler_params=pltpu.CompilerParams(needs_layout_passes=False),
      out_type=jax.ShapeDtypeStruct((B * C, H, W), jnp.float32),
      scratch_types=[
          pltpu.VMEM((CPW * N,), jnp.float32),
          pltpu.VMEM((HB, W), jnp.int32),
          pltpu.VMEM((HB, W), jnp.int32),
          pltpu.VMEM((CPW, HB, W), jnp.float32),
          pltpu.VMEM((CPW, HB, W), jnp.float32),
          pltpu.SemaphoreType.DMA,
          pltpu.SemaphoreType.DMA,
          pltpu.SemaphoreType.DMA,
          pltpu.SemaphoreType.DMA,
      ],
  )
  def k(table_hbm, mask_hbm, out_hbm, tab_v, idx0, idx1, out0, out1,
        si0, si1, so0, so1):
    wid = lax.axis_index("s") * NC + lax.axis_index("c")
    row0 = wid * CPW
    b = row0 // C
    pltpu.sync_copy(table_hbm.at[pl.ds(row0 * N, CPW * N)], tab_v)

    idx_bufs = (idx0, idx1)
    out_bufs = (out0, out1)
    isems = (si0, si1)
    osems = (so0, so1)

    pltpu.async_copy(mask_hbm.at[b, pl.ds(0, HB), :], idx0, si0)
    pltpu.async_copy(mask_hbm.at[b, pl.ds(HB, HB), :], idx1, si1)

    def body(tt, carry):
      for par in range(2):
        t = 2 * tt + par
        hb = pl.multiple_of(t * HB, HB)
        idx_v = idx_bufs[par]
        out_v = out_bufs[par]

        pltpu.make_async_copy(mask_hbm.at[b, pl.ds(0, HB), :], idx_v,
                              isems[par]).wait()

        @pl.when(tt >= 1)
        def _():
          pltpu.make_async_copy(
              out_v, out_hbm.at[pl.ds(0, CPW), pl.ds(0, HB), :],
              osems[par]).wait()

        for h in range(HB):
          def grp(j, carry2, h=h):
            idx = idx_v[h, pl.ds(j * L, L)]
            vals = [
                plsc.load_gather(tab_v.at[pl.ds(cc * N, N)], [idx])
                for cc in range(CPW)
            ]
            for cc in range(CPW):
              out_v[cc, h, pl.ds(j * L, L)] = vals[cc]
            return carry2

          lax.fori_loop(0, GPR, grp, 0, unroll=4)

        @pl.when(t + 2 < NT)
        def _():
          hb2 = pl.multiple_of((t + 2) * HB, HB)
          pltpu.async_copy(mask_hbm.at[b, pl.ds(hb2, HB), :], idx_v,
                           isems[par])

        for cc in range(CPW):
          pltpu.async_copy(
              out_v.at[cc],
              out_hbm.at[row0 + cc, pl.ds(hb, HB), :],
              osems[par])
      return carry

    lax.fori_loop(0, NT // 2, body, 0)

    for par in range(2):
      pltpu.make_async_copy(
          out_bufs[par], out_hbm.at[pl.ds(0, CPW), pl.ds(0, HB), :],
          osems[par]).wait()

  return k(table_flat, mask3)


def kernel(cat_encoded_wg, shape_input_features_in, label_mask, device):
  B, C, N = cat_encoded_wg.shape
  _, _, H, W = label_mask.shape
  out = _unpool(cat_encoded_wg.reshape(-1), label_mask.reshape(B, H, W),
                B, C, N, H, W)
  return out.reshape(B, C, H, W)
